# baseline (speedup 1.0000x reference)
import jax
import jax.numpy as jnp
from jax import lax
from jax.experimental import pallas as pl
from jax.experimental.pallas import tpu as pltpu
from jax.experimental.pallas import tpu_sc as plsc

VOCAB = 100000
EMBED = 64
NC, NS = 2, 16
NW = NC * NS
G = 128
VPAD = 102400
H = 200
BATCH = 16384
BT_PER_W = (BATCH // G) // NW
TPB = H // 8
NT = BT_PER_W * TPB


def _remap_body(idmap_hbm, table_hbm, remap_hbm, idx_v, rows_v, sem):
    wid = lax.axis_index("s") * NC + lax.axis_index("c")
    gpw = VPAD // (G * NW)

    @pl.loop(0, gpw)
    def body(i):
        g = wid * gpw + i
        pltpu.sync_copy(idmap_hbm.at[pl.ds(g * G, G)], idx_v)
        pltpu.async_copy(table_hbm.at[idx_v], rows_v, sem).wait()
        pltpu.sync_copy(rows_v, remap_hbm.at[pl.ds(g * G, G)])


def _gather_body(
    idsT_hbm, remap_hbm, out_hbm,
    ib_v, rows_v, trs_v,
    gsem, wsem0, wsem1,
):
    wid = lax.axis_index("s") * NC + lax.axis_index("c")
    it16 = lax.iota(jnp.int32, 16)
    NB = NT * 8
    z16 = jnp.zeros((16,), jnp.int32)

    def load_idx_tile(T):
        col = (wid * BT_PER_W + T // TPB) * G
        row = lax.rem(T, TPB) * 8
        pltpu.sync_copy(
            idsT_hbm.at[pl.ds(row, 8), pl.ds(col, G)],
            ib_v.at[pl.ds(lax.rem(T, 2) * 8, 8)],
        )

    def fire_gather(j1, p1):
        slot = lax.rem(j1 // 8, 2) * 8 + lax.rem(j1, 8)
        pltpu.async_copy(
            remap_hbm.at[ib_v.at[slot]],
            rows_v.at[pl.ds(p1 * G, G)],
            gsem,
        )

    def gather_wait():
        pltpu.make_async_copy(
            remap_hbm.at[pl.ds(0, G)], rows_v.at[pl.ds(0, G)], gsem
        ).wait()

    def write_drain(sem):
        pltpu.make_async_copy(
            trs_v.at[pl.ds(0, 1)],
            out_hbm.at[pl.ds(0, 1), :, pl.ds(0, 1), :, :],
            sem,
        ).wait()

    load_idx_tile(jnp.int32(0))
    fire_gather(jnp.int32(0), jnp.int32(0))

    @pl.loop(0, NT)
    def outer(T):
        @pl.when(T + 1 < NT)
        def _():
            load_idx_tile(T + 1)

        col_t = wid * BT_PER_W + T // TPB
        hbase = lax.rem(T, TPB) * 8

        @pl.loop(0, 8)
        def inner(dh):
            j = T * 8 + dh
            p = lax.rem(dh, 2)

            gather_wait()

            @pl.when(j + 1 < NB)
            def _():
                fire_gather(j + 1, 1 - p)

            @pl.when((j >= 2) & (p == 0))
            def _():
                write_drain(wsem0)

            @pl.when((j >= 2) & (p == 1))
            def _():
                write_drain(wsem1)

            pG = p * G
            p16 = jnp.full((16,), p, jnp.int32)
            ridx64s = [
                (pG + t0 * 16 + it16) * EMBED for t0 in range(G // 16)
            ]
            tvecs = [t0 * 16 + it16 for t0 in range(G // 16)]

            @plsc.parallel_loop(0, EMBED, unroll=4)
            def tbody(e):
                col = lax.bitwise_and(jnp.full((16,), e, jnp.int32) + it16, 63)
                e8v = lax.shift_right_logical(col, 3)
                elv = lax.bitwise_and(col, 7)
                for t0 in range(G // 16):
                    v = plsc.load_gather(rows_v, [z16, ridx64s[t0] + col])
                    plsc.store_scatter(
                        trs_v, [p16, e8v, z16, elv, tvecs[t0]], v
                    )

            dst = out_hbm.at[pl.ds(hbase + dh, 1), :, pl.ds(col_t, 1), :, :]

            @pl.when(p == 0)
            def _():
                pltpu.async_copy(trs_v.at[pl.ds(0, 1)], dst, wsem0)

            @pl.when(p == 1)
            def _():
                pltpu.async_copy(trs_v.at[pl.ds(1, 1)], dst, wsem1)

    write_drain(wsem0)
    write_drain(wsem1)


def kernel(input_ids, id_map, table):
    B, HH = input_ids.shape
    idsT = input_ids.astype(jnp.int32).T
    idm = id_map.astype(jnp.int32)
    idm = jnp.concatenate([idm, jnp.zeros((VPAD - VOCAB,), jnp.int32)])
    table = table.astype(jnp.float32)

    mesh = plsc.VectorSubcoreMesh(core_axis_name="c", subcore_axis_name="s")
    params = pltpu.CompilerParams(
        use_tc_tiling_on_sc=False, needs_layout_passes=False
    )

    remap = pl.kernel(
        _remap_body,
        out_type=jax.ShapeDtypeStruct((VPAD, EMBED), jnp.float32),
        mesh=mesh,
        compiler_params=params,
        scratch_types=[
            pltpu.VMEM((G,), jnp.int32),
            pltpu.VMEM((G, EMBED), jnp.float32),
            pltpu.SemaphoreType.DMA,
        ],
        name="remap_table_sc",
    )(idm, table)

    out5 = pl.kernel(
        _gather_body,
        out_type=jax.ShapeDtypeStruct((H, 8, BATCH // G, 8, G), jnp.float32),
        mesh=mesh,
        compiler_params=params,
        scratch_types=[
            pltpu.VMEM((16, G), jnp.int32),
            pltpu.VMEM((2 * G, EMBED), jnp.float32),
            pltpu.VMEM((2, 8, 1, 8, G), jnp.float32),
            pltpu.SemaphoreType.DMA,
            pltpu.SemaphoreType.DMA,
            pltpu.SemaphoreType.DMA,
        ],
        name="token_gather_sc",
    )(idsT, remap)

    return out5.transpose(2, 4, 0, 1, 3).reshape(B, HH, EMBED)


# speedup vs baseline: 1.3949x; 1.3949x over previous
import jax
import jax.numpy as jnp
from jax import lax
from jax.experimental import pallas as pl
from jax.experimental.pallas import tpu as pltpu
from jax.experimental.pallas import tpu_sc as plsc

VOCAB = 100000
EMBED = 64
NC, NS = 2, 16
NW = NC * NS
G = 128
VPAD = 102400
H = 200
BATCH = 16384
BT_PER_W = (BATCH // G) // NW
TPB = H // 8
NT = BT_PER_W * TPB


def _remap_body(idmap_hbm, table_hbm, remap_hbm, idx_v, rows_v, sem):
    wid = lax.axis_index("s") * NC + lax.axis_index("c")
    gpw = VPAD // (G * NW)

    @pl.loop(0, gpw)
    def body(i):
        g = wid * gpw + i
        pltpu.sync_copy(idmap_hbm.at[pl.ds(g * G, G)], idx_v)
        pltpu.async_copy(table_hbm.at[idx_v], rows_v, sem).wait()
        pltpu.sync_copy(rows_v, remap_hbm.at[pl.ds(g * G, G)])


def _gather_body(
    idsT_hbm, remap_hbm, out_hbm,
    ib_v, rows_v, trs_v,
    gsem, wsem0, wsem1,
):
    wid = lax.axis_index("s") * NC + lax.axis_index("c")
    it16 = lax.iota(jnp.int32, 16)
    NB = NT * 8
    z16 = jnp.zeros((16,), jnp.int32)

    def load_idx_tile(T):
        col = (wid * BT_PER_W + T // TPB) * G
        row = lax.rem(T, TPB) * 8
        pltpu.sync_copy(
            idsT_hbm.at[pl.ds(row, 8), pl.ds(col, G)],
            ib_v.at[pl.ds(lax.rem(T, 2) * 8, 8)],
        )

    def fire_gather(j1, p1):
        slot = lax.rem(j1 // 8, 2) * 8 + lax.rem(j1, 8)
        pltpu.async_copy(
            remap_hbm.at[ib_v.at[slot]],
            rows_v.at[pl.ds(p1 * G, G)],
            gsem,
        )

    def gather_wait():
        pltpu.make_async_copy(
            remap_hbm.at[pl.ds(0, G)], rows_v.at[pl.ds(0, G)], gsem
        ).wait()

    def write_drain(sem):
        pltpu.make_async_copy(
            trs_v.at[pl.ds(0, 1)],
            out_hbm.at[pl.ds(0, 1), :, pl.ds(0, 1), :, :],
            sem,
        ).wait()

    load_idx_tile(jnp.int32(0))
    fire_gather(jnp.int32(0), jnp.int32(0))
    fire_gather(jnp.int32(1), jnp.int32(1))

    @pl.loop(0, NT)
    def outer(T):
        @pl.when(T + 1 < NT)
        def _():
            load_idx_tile(T + 1)

        col_t = wid * BT_PER_W + T // TPB
        hbase = lax.rem(T, TPB) * 8

        @pl.loop(0, 8)
        def inner(dh):
            j = T * 8 + dh
            p = lax.rem(dh, 2)

            gather_wait()

            @pl.when(j + 2 < NB)
            def _():
                fire_gather(j + 2, lax.rem(dh + 2, 4))

            @pl.when((j >= 2) & (p == 0))
            def _():
                write_drain(wsem0)

            @pl.when((j >= 2) & (p == 1))
            def _():
                write_drain(wsem1)

            pG = lax.rem(dh, 4) * G
            p16 = jnp.full((16,), p, jnp.int32)
            ridx64s = [
                (pG + t0 * 16 + it16) * EMBED for t0 in range(G // 16)
            ]
            tvecs = [t0 * 16 + it16 for t0 in range(G // 16)]

            @plsc.parallel_loop(0, EMBED, unroll=4)
            def tbody(e):
                col = lax.bitwise_and(jnp.full((16,), e, jnp.int32) + it16, 63)
                e8v = lax.shift_right_logical(col, 3)
                elv = lax.bitwise_and(col, 7)
                for t0 in range(G // 16):
                    v = plsc.load_gather(rows_v, [z16, ridx64s[t0] + col])
                    plsc.store_scatter(
                        trs_v, [p16, e8v, z16, elv, tvecs[t0]], v
                    )

            dst = out_hbm.at[pl.ds(hbase + dh, 1), :, pl.ds(col_t, 1), :, :]

            @pl.when(p == 0)
            def _():
                pltpu.async_copy(trs_v.at[pl.ds(0, 1)], dst, wsem0)

            @pl.when(p == 1)
            def _():
                pltpu.async_copy(trs_v.at[pl.ds(1, 1)], dst, wsem1)

    write_drain(wsem0)
    write_drain(wsem1)


def kernel(input_ids, id_map, table):
    B, HH = input_ids.shape
    idsT = input_ids.astype(jnp.int32).T
    idm = id_map.astype(jnp.int32)
    idm = jnp.concatenate([idm, jnp.zeros((VPAD - VOCAB,), jnp.int32)])
    table = table.astype(jnp.float32)

    mesh = plsc.VectorSubcoreMesh(core_axis_name="c", subcore_axis_name="s")
    params = pltpu.CompilerParams(
        use_tc_tiling_on_sc=False, needs_layout_passes=False
    )

    remap = pl.kernel(
        _remap_body,
        out_type=jax.ShapeDtypeStruct((VPAD, EMBED), jnp.float32),
        mesh=mesh,
        compiler_params=params,
        scratch_types=[
            pltpu.VMEM((G,), jnp.int32),
            pltpu.VMEM((G, EMBED), jnp.float32),
            pltpu.SemaphoreType.DMA,
        ],
        name="remap_table_sc",
    )(idm, table)

    out5 = pl.kernel(
        _gather_body,
        out_type=jax.ShapeDtypeStruct((H, 8, BATCH // G, 8, G), jnp.float32),
        mesh=mesh,
        compiler_params=params,
        scratch_types=[
            pltpu.VMEM((16, G), jnp.int32),
            pltpu.VMEM((4 * G, EMBED), jnp.float32),
            pltpu.VMEM((2, 8, 1, 8, G), jnp.float32),
            pltpu.SemaphoreType.DMA,
            pltpu.SemaphoreType.DMA,
            pltpu.SemaphoreType.DMA,
        ],
        name="token_gather_sc",
    )(idsT, remap)

    return out5.transpose(2, 4, 0, 1, 3).reshape(B, HH, EMBED)


# speedup vs baseline: 1.4679x; 1.0523x over previous
import jax
import jax.numpy as jnp
from jax import lax
from jax.experimental import pallas as pl
from jax.experimental.pallas import tpu as pltpu
from jax.experimental.pallas import tpu_sc as plsc

VOCAB = 100000
EMBED = 64
NC, NS = 2, 16
NW = NC * NS
G = 128
VPAD = 102400
H = 200
BATCH = 16384
BT_PER_W = (BATCH // G) // NW
TPB = H // 8
NT = BT_PER_W * TPB


def _remap_body(idmap_hbm, table_hbm, remap_hbm, idx_v, rows_v, sem):
    wid = lax.axis_index("s") * NC + lax.axis_index("c")
    gpw = VPAD // (G * NW)

    @pl.loop(0, gpw)
    def body(i):
        g = wid * gpw + i
        pltpu.sync_copy(idmap_hbm.at[pl.ds(g * G, G)], idx_v)
        pltpu.async_copy(table_hbm.at[idx_v], rows_v, sem).wait()
        pltpu.sync_copy(rows_v, remap_hbm.at[pl.ds(g * G, G)])


def _gather_body(
    idsT_hbm, remap_hbm, out_hbm,
    ib_v, rows_v, trs_v,
    gsem, wsem0, wsem1,
):
    wid = lax.axis_index("s") * NC + lax.axis_index("c")
    it16 = lax.iota(jnp.int32, 16)
    NB = NT * 8
    z16 = jnp.zeros((16,), jnp.int32)

    def load_idx_tile(T):
        col = (wid * BT_PER_W + T // TPB) * G
        row = lax.rem(T, TPB) * 8
        pltpu.sync_copy(
            idsT_hbm.at[pl.ds(row, 8), pl.ds(col, G)],
            ib_v.at[pl.ds(lax.rem(T, 2) * 8, 8)],
        )

    def fire_gather(j1, p1):
        slot = lax.rem(j1 // 8, 2) * 8 + lax.rem(j1, 8)
        pltpu.async_copy(
            remap_hbm.at[ib_v.at[slot]],
            rows_v.at[pl.ds(p1 * G, G)],
            gsem,
        )

    def gather_wait():
        pltpu.make_async_copy(
            remap_hbm.at[pl.ds(0, G)], rows_v.at[pl.ds(0, G)], gsem
        ).wait()

    def write_drain(sem):
        pltpu.make_async_copy(
            trs_v.at[pl.ds(0, 1)],
            out_hbm.at[pl.ds(0, 1), :, pl.ds(0, 1), :, :],
            sem,
        ).wait()

    load_idx_tile(jnp.int32(0))
    fire_gather(jnp.int32(0), jnp.int32(0))
    fire_gather(jnp.int32(1), jnp.int32(1))
    fire_gather(jnp.int32(2), jnp.int32(2))

    @pl.loop(0, NT)
    def outer(T):
        @pl.when(T + 1 < NT)
        def _():
            load_idx_tile(T + 1)

        col_t = wid * BT_PER_W + T // TPB
        hbase = lax.rem(T, TPB) * 8

        @pl.loop(0, 8)
        def inner(dh):
            j = T * 8 + dh
            p = lax.rem(dh, 2)

            gather_wait()

            @pl.when(j + 3 < NB)
            def _():
                fire_gather(j + 3, lax.rem(dh + 3, 4))

            @pl.when((j >= 2) & (p == 0))
            def _():
                write_drain(wsem0)

            @pl.when((j >= 2) & (p == 1))
            def _():
                write_drain(wsem1)

            pG = lax.rem(dh, 4) * G
            p16 = jnp.full((16,), p, jnp.int32)
            ridx64s = [
                (pG + t0 * 16 + it16) * EMBED for t0 in range(G // 16)
            ]
            tvecs = [t0 * 16 + it16 for t0 in range(G // 16)]

            @plsc.parallel_loop(0, EMBED, unroll=4)
            def tbody(e):
                col = lax.bitwise_and(jnp.full((16,), e, jnp.int32) + it16, 63)
                e8v = lax.shift_right_logical(col, 3)
                elv = lax.bitwise_and(col, 7)
                for t0 in range(G // 16):
                    v = plsc.load_gather(rows_v, [z16, ridx64s[t0] + col])
                    plsc.store_scatter(
                        trs_v, [p16, e8v, z16, elv, tvecs[t0]], v
                    )

            dst = out_hbm.at[pl.ds(hbase + dh, 1), :, pl.ds(col_t, 1), :, :]

            @pl.when(p == 0)
            def _():
                pltpu.async_copy(trs_v.at[pl.ds(0, 1)], dst, wsem0)

            @pl.when(p == 1)
            def _():
                pltpu.async_copy(trs_v.at[pl.ds(1, 1)], dst, wsem1)

    write_drain(wsem0)
    write_drain(wsem1)


def kernel(input_ids, id_map, table):
    B, HH = input_ids.shape
    idsT = input_ids.astype(jnp.int32).T
    idm = id_map.astype(jnp.int32)
    idm = jnp.concatenate([idm, jnp.zeros((VPAD - VOCAB,), jnp.int32)])
    table = table.astype(jnp.float32)

    mesh = plsc.VectorSubcoreMesh(core_axis_name="c", subcore_axis_name="s")
    params = pltpu.CompilerParams(
        use_tc_tiling_on_sc=False, needs_layout_passes=False
    )

    remap = pl.kernel(
        _remap_body,
        out_type=jax.ShapeDtypeStruct((VPAD, EMBED), jnp.float32),
        mesh=mesh,
        compiler_params=params,
        scratch_types=[
            pltpu.VMEM((G,), jnp.int32),
            pltpu.VMEM((G, EMBED), jnp.float32),
            pltpu.SemaphoreType.DMA,
        ],
        name="remap_table_sc",
    )(idm, table)

    out5 = pl.kernel(
        _gather_body,
        out_type=jax.ShapeDtypeStruct((H, 8, BATCH // G, 8, G), jnp.float32),
        mesh=mesh,
        compiler_params=params,
        scratch_types=[
            pltpu.VMEM((16, G), jnp.int32),
            pltpu.VMEM((4 * G, EMBED), jnp.float32),
            pltpu.VMEM((2, 8, 1, 8, G), jnp.float32),
            pltpu.SemaphoreType.DMA,
            pltpu.SemaphoreType.DMA,
            pltpu.SemaphoreType.DMA,
        ],
        name="token_gather_sc",
    )(idsT, remap)

    return out5.transpose(2, 4, 0, 1, 3).reshape(B, HH, EMBED)


# speedup vs baseline: 1.4877x; 1.0135x over previous
import jax
import jax.numpy as jnp
from jax import lax
from jax.experimental import pallas as pl
from jax.experimental.pallas import tpu as pltpu
from jax.experimental.pallas import tpu_sc as plsc

VOCAB = 100000
EMBED = 64
NC, NS = 2, 16
NW = NC * NS
G = 128
VPAD = 102400
H = 200
BATCH = 16384
BT_PER_W = (BATCH // G) // NW
TPB = H // 8
NT = BT_PER_W * TPB


def _remap_body(idmap_hbm, table_hbm, remap_hbm, idx_v, rows_v, sem):
    wid = lax.axis_index("s") * NC + lax.axis_index("c")
    gpw = VPAD // (G * NW)

    @pl.loop(0, gpw)
    def body(i):
        g = wid * gpw + i
        pltpu.sync_copy(idmap_hbm.at[pl.ds(g * G, G)], idx_v)
        pltpu.async_copy(table_hbm.at[idx_v], rows_v, sem).wait()
        pltpu.sync_copy(rows_v, remap_hbm.at[pl.ds(g * G, G)])


def _gather_body(
    idsT_hbm, remap_hbm, out_hbm,
    ib_v, rows_v, trs_v,
    gsem, wsem0, wsem1, isem,
):
    wid = lax.axis_index("s") * NC + lax.axis_index("c")
    it16 = lax.iota(jnp.int32, 16)
    NB = NT * 8
    z16 = jnp.zeros((16,), jnp.int32)

    def load_idx_tile(T):
        col = (wid * BT_PER_W + T // TPB) * G
        row = lax.rem(T, TPB) * 8
        pltpu.async_copy(
            idsT_hbm.at[pl.ds(row, 8), pl.ds(col, G)],
            ib_v.at[pl.ds(lax.rem(T, 2) * 8, 8)],
            isem,
        )

    def idx_wait():
        pltpu.make_async_copy(
            idsT_hbm.at[pl.ds(0, 8), pl.ds(0, G)], ib_v.at[pl.ds(0, 8)], isem
        ).wait()

    def fire_gather(j1, p1):
        slot = lax.rem(j1 // 8, 2) * 8 + lax.rem(j1, 8)
        pltpu.async_copy(
            remap_hbm.at[ib_v.at[slot]],
            rows_v.at[pl.ds(p1 * G, G)],
            gsem,
        )

    def gather_wait():
        pltpu.make_async_copy(
            remap_hbm.at[pl.ds(0, G)], rows_v.at[pl.ds(0, G)], gsem
        ).wait()

    def write_drain(sem):
        pltpu.make_async_copy(
            trs_v.at[pl.ds(0, 1)],
            out_hbm.at[pl.ds(0, 1), :, pl.ds(0, 1), :, :],
            sem,
        ).wait()

    load_idx_tile(jnp.int32(0))
    idx_wait()
    fire_gather(jnp.int32(0), jnp.int32(0))
    fire_gather(jnp.int32(1), jnp.int32(1))
    fire_gather(jnp.int32(2), jnp.int32(2))

    @pl.loop(0, NT)
    def outer(T):
        @pl.when(T + 1 < NT)
        def _():
            load_idx_tile(T + 1)

        col_t = wid * BT_PER_W + T // TPB
        hbase = lax.rem(T, TPB) * 8

        @pl.loop(0, 8)
        def inner(dh):
            j = T * 8 + dh
            p = lax.rem(dh, 2)

            gather_wait()

            @pl.when((dh == 5) & (T + 1 < NT))
            def _():
                idx_wait()

            @pl.when(j + 3 < NB)
            def _():
                fire_gather(j + 3, lax.rem(dh + 3, 4))

            @pl.when((j >= 2) & (p == 0))
            def _():
                write_drain(wsem0)

            @pl.when((j >= 2) & (p == 1))
            def _():
                write_drain(wsem1)

            pG = lax.rem(dh, 4) * G
            p16 = jnp.full((16,), p, jnp.int32)
            ridx64s = [
                (pG + t0 * 16 + it16) * EMBED for t0 in range(G // 16)
            ]
            tvecs = [t0 * 16 + it16 for t0 in range(G // 16)]

            @plsc.parallel_loop(0, EMBED, unroll=4)
            def tbody(e):
                col = lax.bitwise_and(jnp.full((16,), e, jnp.int32) + it16, 63)
                e8v = lax.shift_right_logical(col, 3)
                elv = lax.bitwise_and(col, 7)
                for t0 in range(G // 16):
                    v = plsc.load_gather(rows_v, [z16, ridx64s[t0] + col])
                    plsc.store_scatter(
                        trs_v, [p16, e8v, z16, elv, tvecs[t0]], v
                    )

            dst = out_hbm.at[pl.ds(hbase + dh, 1), :, pl.ds(col_t, 1), :, :]

            @pl.when(p == 0)
            def _():
                pltpu.async_copy(trs_v.at[pl.ds(0, 1)], dst, wsem0)

            @pl.when(p == 1)
            def _():
                pltpu.async_copy(trs_v.at[pl.ds(1, 1)], dst, wsem1)

    write_drain(wsem0)
    write_drain(wsem1)


def kernel(input_ids, id_map, table):
    B, HH = input_ids.shape
    idsT = input_ids.astype(jnp.int32).T
    idm = id_map.astype(jnp.int32)
    idm = jnp.concatenate([idm, jnp.zeros((VPAD - VOCAB,), jnp.int32)])
    table = table.astype(jnp.float32)

    mesh = plsc.VectorSubcoreMesh(core_axis_name="c", subcore_axis_name="s")
    params = pltpu.CompilerParams(
        use_tc_tiling_on_sc=False, needs_layout_passes=False
    )

    remap = pl.kernel(
        _remap_body,
        out_type=jax.ShapeDtypeStruct((VPAD, EMBED), jnp.float32),
        mesh=mesh,
        compiler_params=params,
        scratch_types=[
            pltpu.VMEM((G,), jnp.int32),
            pltpu.VMEM((G, EMBED), jnp.float32),
            pltpu.SemaphoreType.DMA,
        ],
        name="remap_table_sc",
    )(idm, table)

    out5 = pl.kernel(
        _gather_body,
        out_type=jax.ShapeDtypeStruct((H, 8, BATCH // G, 8, G), jnp.float32),
        mesh=mesh,
        compiler_params=params,
        scratch_types=[
            pltpu.VMEM((16, G), jnp.int32),
            pltpu.VMEM((4 * G, EMBED), jnp.float32),
            pltpu.VMEM((2, 8, 1, 8, G), jnp.float32),
            pltpu.SemaphoreType.DMA,
            pltpu.SemaphoreType.DMA,
            pltpu.SemaphoreType.DMA,
            pltpu.SemaphoreType.DMA,
        ],
        name="token_gather_sc",
    )(idsT, remap)

    return out5.transpose(2, 4, 0, 1, 3).reshape(B, HH, EMBED)


# speedup vs baseline: 1.5153x; 1.0185x over previous
import jax
import jax.numpy as jnp
from jax import lax
from jax.experimental import pallas as pl
from jax.experimental.pallas import tpu as pltpu
from jax.experimental.pallas import tpu_sc as plsc

VOCAB = 100000
EMBED = 64
NC, NS = 2, 16
NW = NC * NS
G = 128
VPAD = 102400
H = 200
BATCH = 16384
BT_PER_W = (BATCH // G) // NW
TPB = H // 8
NT = BT_PER_W * TPB


def _remap_body(idmap_hbm, table_hbm, remap_hbm, idx_v, rows_v, gsem, w0, w1):
    wid = lax.axis_index("s") * NC + lax.axis_index("c")
    gpw = VPAD // (G * NW)
    base_g = wid * gpw

    def load_idx(i):
        pltpu.sync_copy(
            idmap_hbm.at[pl.ds((base_g + i) * G, G)],
            idx_v.at[pl.ds(lax.rem(i, 2) * G, G)],
        )

    def fire(i):
        pltpu.async_copy(
            table_hbm.at[idx_v.at[pl.ds(lax.rem(i, 2) * G, G)]],
            rows_v.at[pl.ds(lax.rem(i, 2) * G, G)],
            gsem,
        )

    def gwait():
        pltpu.make_async_copy(
            table_hbm.at[pl.ds(0, G)], rows_v.at[pl.ds(0, G)], gsem
        ).wait()

    def wdrain(s):
        pltpu.make_async_copy(
            rows_v.at[pl.ds(0, G)], remap_hbm.at[pl.ds(0, G)], s
        ).wait()

    load_idx(jnp.int32(0))
    fire(jnp.int32(0))

    @pl.loop(0, gpw)
    def body(i):
        p = lax.rem(i, 2)

        @pl.when(i + 1 < gpw)
        def _():
            load_idx(i + 1)

        gwait()

        @pl.when((i >= 1) & (i + 1 < gpw) & (p == 0))
        def _():
            wdrain(w1)

        @pl.when((i >= 1) & (i + 1 < gpw) & (p == 1))
        def _():
            wdrain(w0)

        @pl.when(i + 1 < gpw)
        def _():
            fire(i + 1)

        dst = remap_hbm.at[pl.ds((base_g + i) * G, G)]

        @pl.when(p == 0)
        def _():
            pltpu.async_copy(rows_v.at[pl.ds(0, G)], dst, w0)

        @pl.when(p == 1)
        def _():
            pltpu.async_copy(rows_v.at[pl.ds(G, G)], dst, w1)

    wdrain(w0)
    wdrain(w1)


def _gather_body(
    idsT_hbm, remap_hbm, out_hbm,
    ib_v, rows_v, trs_v,
    gsem, wsem0, wsem1, isem,
):
    wid = lax.axis_index("s") * NC + lax.axis_index("c")
    it16 = lax.iota(jnp.int32, 16)
    NB = NT * 8
    z16 = jnp.zeros((16,), jnp.int32)

    def load_idx_tile(T):
        col = (wid * BT_PER_W + T // TPB) * G
        row = lax.rem(T, TPB) * 8
        pltpu.async_copy(
            idsT_hbm.at[pl.ds(row, 8), pl.ds(col, G)],
            ib_v.at[pl.ds(lax.rem(T, 2) * 8, 8)],
            isem,
        )

    def idx_wait():
        pltpu.make_async_copy(
            idsT_hbm.at[pl.ds(0, 8), pl.ds(0, G)], ib_v.at[pl.ds(0, 8)], isem
        ).wait()

    def fire_gather(j1, p1):
        slot = lax.rem(j1 // 8, 2) * 8 + lax.rem(j1, 8)
        pltpu.async_copy(
            remap_hbm.at[ib_v.at[slot]],
            rows_v.at[pl.ds(p1 * G, G)],
            gsem,
        )

    def gather_wait():
        pltpu.make_async_copy(
            remap_hbm.at[pl.ds(0, G)], rows_v.at[pl.ds(0, G)], gsem
        ).wait()

    def write_drain(sem):
        pltpu.make_async_copy(
            trs_v.at[pl.ds(0, 1)],
            out_hbm.at[pl.ds(0, 1), :, pl.ds(0, 1), :, :],
            sem,
        ).wait()

    load_idx_tile(jnp.int32(0))
    idx_wait()
    fire_gather(jnp.int32(0), jnp.int32(0))
    fire_gather(jnp.int32(1), jnp.int32(1))
    fire_gather(jnp.int32(2), jnp.int32(2))

    @pl.loop(0, NT)
    def outer(T):
        @pl.when(T + 1 < NT)
        def _():
            load_idx_tile(T + 1)

        col_t = wid * BT_PER_W + T // TPB
        hbase = lax.rem(T, TPB) * 8

        @pl.loop(0, 8)
        def inner(dh):
            j = T * 8 + dh
            p = lax.rem(dh, 2)

            gather_wait()

            @pl.when((dh == 5) & (T + 1 < NT))
            def _():
                idx_wait()

            @pl.when(j + 3 < NB)
            def _():
                fire_gather(j + 3, lax.rem(dh + 3, 4))

            @pl.when((j >= 2) & (p == 0))
            def _():
                write_drain(wsem0)

            @pl.when((j >= 2) & (p == 1))
            def _():
                write_drain(wsem1)

            pG = lax.rem(dh, 4) * G
            p16 = jnp.full((16,), p, jnp.int32)
            ridx64s = [
                (pG + t0 * 16 + it16) * EMBED for t0 in range(G // 16)
            ]
            tvecs = [t0 * 16 + it16 for t0 in range(G // 16)]

            @plsc.parallel_loop(0, EMBED, unroll=4)
            def tbody(e):
                col = lax.bitwise_and(jnp.full((16,), e, jnp.int32) + it16, 63)
                e8v = lax.shift_right_logical(col, 3)
                elv = lax.bitwise_and(col, 7)
                for t0 in range(G // 16):
                    v = plsc.load_gather(rows_v, [z16, ridx64s[t0] + col])
                    plsc.store_scatter(
                        trs_v, [p16, e8v, z16, elv, tvecs[t0]], v
                    )

            dst = out_hbm.at[pl.ds(hbase + dh, 1), :, pl.ds(col_t, 1), :, :]

            @pl.when(p == 0)
            def _():
                pltpu.async_copy(trs_v.at[pl.ds(0, 1)], dst, wsem0)

            @pl.when(p == 1)
            def _():
                pltpu.async_copy(trs_v.at[pl.ds(1, 1)], dst, wsem1)

    write_drain(wsem0)
    write_drain(wsem1)


def kernel(input_ids, id_map, table):
    B, HH = input_ids.shape
    idsT = input_ids.astype(jnp.int32).T
    idm = id_map.astype(jnp.int32)
    idm = jnp.concatenate([idm, jnp.zeros((VPAD - VOCAB,), jnp.int32)])
    table = table.astype(jnp.float32)

    mesh = plsc.VectorSubcoreMesh(core_axis_name="c", subcore_axis_name="s")
    params = pltpu.CompilerParams(
        use_tc_tiling_on_sc=False, needs_layout_passes=False
    )

    remap = pl.kernel(
        _remap_body,
        out_type=jax.ShapeDtypeStruct((VPAD, EMBED), jnp.float32),
        mesh=mesh,
        compiler_params=params,
        scratch_types=[
            pltpu.VMEM((2 * G,), jnp.int32),
            pltpu.VMEM((2 * G, EMBED), jnp.float32),
            pltpu.SemaphoreType.DMA,
            pltpu.SemaphoreType.DMA,
            pltpu.SemaphoreType.DMA,
        ],
        name="remap_table_sc",
    )(idm, table)

    out5 = pl.kernel(
        _gather_body,
        out_type=jax.ShapeDtypeStruct((H, 8, BATCH // G, 8, G), jnp.float32),
        mesh=mesh,
        compiler_params=params,
        scratch_types=[
            pltpu.VMEM((16, G), jnp.int32),
            pltpu.VMEM((4 * G, EMBED), jnp.float32),
            pltpu.VMEM((2, 8, 1, 8, G), jnp.float32),
            pltpu.SemaphoreType.DMA,
            pltpu.SemaphoreType.DMA,
            pltpu.SemaphoreType.DMA,
            pltpu.SemaphoreType.DMA,
        ],
        name="token_gather_sc",
    )(idsT, remap)

    return out5.transpose(2, 4, 0, 1, 3).reshape(B, HH, EMBED)
